# baseline (speedup 1.0000x reference)
import jax
import jax.numpy as jnp
from jax import lax
from jax.experimental import pallas as pl
from jax.experimental.pallas import tpu as pltpu
from jax.experimental.pallas import tpu_sc as plsc

N_ROWS = 1_000_000
D = 32
B = 16384

NC, NS, L = 2, 16, 16
NW = NC * NS
BPW = B // NW
CHUNK = 128
NCH = BPW // CHUNK
NGRP = BPW // L

ROWS_PER_BLK = 8000
NBLK = N_ROWS // ROWS_PER_BLK


def _sc_body(z_hbm, y_hbm, u_hbm, i_hbm, r_hbm, out_hbm,
             u_v, i_v, r_v, zu, zi, yu, yi, out_v, sem):
    wid = lax.axis_index("s") * NC + lax.axis_index("c")

    pltpu.sync_copy(u_hbm.at[pl.ds(wid * NCH, NCH)], u_v)
    pltpu.sync_copy(i_hbm.at[pl.ds(wid * NCH, NCH)], i_v)
    pltpu.sync_copy(r_hbm.at[pl.ds(wid * BPW, BPW)], r_v)

    copies = []
    for c in range(NCH):
        sl = pl.ds(c * CHUNK, CHUNK)
        copies.append(pltpu.async_copy(z_hbm.at[u_v.at[c]], zu.at[sl], sem))
        copies.append(pltpu.async_copy(z_hbm.at[i_v.at[c]], zi.at[sl], sem))
        copies.append(pltpu.async_copy(y_hbm.at[u_v.at[c]], yu.at[sl], sem))
        copies.append(pltpu.async_copy(y_hbm.at[i_v.at[c]], yi.at[sl], sem))
    for cp in copies:
        cp.wait()

    lanes = lax.iota(jnp.int32, L)

    def group(g, acc):
        rows = lanes + g * L
        rhat = jnp.zeros((L,), jnp.float32)
        for d in range(D):
            cols = jnp.full((L,), d, jnp.int32)
            zug = plsc.load_gather(zu, [rows, cols])
            zig = plsc.load_gather(zi, [rows, cols])
            yug = plsc.load_gather(yu, [rows, cols])
            yig = plsc.load_gather(yi, [rows, cols])
            rhat = rhat + zug * zig + yug * yig
        rv = r_v[pl.ds(g * L, L)]
        err = rv - rhat
        return acc + err * err

    acc = lax.fori_loop(0, NGRP, group, jnp.zeros((L,), jnp.float32))
    out_v[...] = acc
    pltpu.sync_copy(out_v, out_hbm.at[wid])


def _build_sc_partials():
    return pl.kernel(
        _sc_body,
        out_type=jax.ShapeDtypeStruct((NW, L), jnp.float32),
        mesh=plsc.VectorSubcoreMesh(core_axis_name="c", subcore_axis_name="s",
                                    num_cores=NC, num_subcores=NS),
        compiler_params=pltpu.CompilerParams(needs_layout_passes=False,
                                             use_tc_tiling_on_sc=False),
        scratch_types=[
            pltpu.VMEM((NCH, CHUNK), jnp.int32),
            pltpu.VMEM((NCH, CHUNK), jnp.int32),
            pltpu.VMEM((BPW,), jnp.float32),
            pltpu.VMEM((BPW, D), jnp.float32),
            pltpu.VMEM((BPW, D), jnp.float32),
            pltpu.VMEM((BPW, D), jnp.float32),
            pltpu.VMEM((BPW, D), jnp.float32),
            pltpu.VMEM((L,), jnp.float32),
            pltpu.SemaphoreType.DMA,
        ],
    )


def _tc_body(y_ref, p_ref, o_ref):
    @pl.when(pl.program_id(0) == 0)
    def _():
        o_ref[0, 0] = jnp.sum(p_ref[...])

    yv = y_ref[...]
    o_ref[0, 0] += jnp.sum(yv * yv)


_tc_loss = pl.pallas_call(
    _tc_body,
    grid=(NBLK,),
    in_specs=[
        pl.BlockSpec((ROWS_PER_BLK, D), lambda b: (b, 0)),
        pl.BlockSpec((NW, L), lambda b: (0, 0)),
    ],
    out_specs=pl.BlockSpec(memory_space=pltpu.SMEM),
    out_shape=jax.ShapeDtypeStruct((1, 1), jnp.float32),
)


def kernel(Z, Y, interaction):
    u = interaction[:, 0].reshape(NW * NCH, CHUNK)
    i = interaction[:, 1].reshape(NW * NCH, CHUNK)
    r = interaction[:, 2].astype(jnp.float32)
    partials = _build_sc_partials()(Z, Y, u, i, r)
    loss = _tc_loss(Y, partials)
    return loss[0, 0]


# speedup vs baseline: 1.2668x; 1.2668x over previous
import jax
import jax.numpy as jnp
from jax import lax
from jax.experimental import pallas as pl
from jax.experimental.pallas import tpu as pltpu
from jax.experimental.pallas import tpu_sc as plsc

N_ROWS = 1_000_000
D = 32
B = 16384
PACK = 4
NP_ROWS = N_ROWS // PACK

NC, NS, L = 2, 16, 16
NW = NC * NS
BPW = B // NW
CHUNK = 128
NCH = BPW // CHUNK
GPC = CHUNK // L

CBLK = 16384
NBLK = -(-N_ROWS // CBLK)


def _sc_body(zp_hbm, yp_hbm, u_hbm, i_hbm, r_hbm, out_hbm,
             u_v, i_v, pu_v, pi_v, r_v, zu, zi, yu, yi, out_v, sem):
    wid = lax.axis_index("s") * NC + lax.axis_index("c")

    pltpu.sync_copy(u_hbm.at[pl.ds(wid * NCH, NCH)], u_v)
    pltpu.sync_copy(i_hbm.at[pl.ds(wid * NCH, NCH)], i_v)
    pltpu.sync_copy(r_hbm.at[pl.ds(wid * BPW, BPW)], r_v)

    for c in range(NCH):
        for q in range(CHUNK // L):
            sl = pl.ds(q * L, L)
            pu_v[c, sl] = jax.lax.shift_right_logical(u_v[c, sl], 2)
            pi_v[c, sl] = jax.lax.shift_right_logical(i_v[c, sl], 2)

    lanes = lax.iota(jnp.int32, L)
    acc = jnp.zeros((L,), jnp.float32)

    for c in range(NCH):
        cz_u = pltpu.async_copy(zp_hbm.at[pu_v.at[c]], zu, sem)
        cz_i = pltpu.async_copy(zp_hbm.at[pi_v.at[c]], zi, sem)
        cy_u = pltpu.async_copy(yp_hbm.at[pu_v.at[c]], yu, sem)
        cy_i = pltpu.async_copy(yp_hbm.at[pi_v.at[c]], yi, sem)
        cz_u.wait()
        cz_i.wait()
        cy_u.wait()
        cy_i.wait()

        def group(g, a, c=c):
            rows = lanes + g * L
            gsl = pl.ds(g * L, L)
            ubase = jax.lax.shift_left(u_v[c, gsl] & 3, 5)
            ibase = jax.lax.shift_left(i_v[c, gsl] & 3, 5)
            rhat = jnp.zeros((L,), jnp.float32)
            for d in range(D):
                ucols = ubase + d
                icols = ibase + d
                zug = plsc.load_gather(zu, [rows, ucols])
                zig = plsc.load_gather(zi, [rows, icols])
                yug = plsc.load_gather(yu, [rows, ucols])
                yig = plsc.load_gather(yi, [rows, icols])
                rhat = rhat + zug * zig + yug * yig
            rv = r_v[pl.ds(c * CHUNK + g * L, L)]
            err = rv - rhat
            return a + err * err

        acc = lax.fori_loop(0, GPC, group, acc)

    out_v[...] = acc
    pltpu.sync_copy(out_v, out_hbm.at[wid])


def _build_sc_partials():
    return pl.kernel(
        _sc_body,
        out_type=jax.ShapeDtypeStruct((NW, L), jnp.float32),
        mesh=plsc.VectorSubcoreMesh(core_axis_name="c", subcore_axis_name="s",
                                    num_cores=NC, num_subcores=NS),
        compiler_params=pltpu.CompilerParams(needs_layout_passes=False,
                                             use_tc_tiling_on_sc=False),
        scratch_types=[
            pltpu.VMEM((NCH, CHUNK), jnp.int32),
            pltpu.VMEM((NCH, CHUNK), jnp.int32),
            pltpu.VMEM((NCH, CHUNK), jnp.int32),
            pltpu.VMEM((NCH, CHUNK), jnp.int32),
            pltpu.VMEM((BPW,), jnp.float32),
            pltpu.VMEM((CHUNK, 128), jnp.float32),
            pltpu.VMEM((CHUNK, 128), jnp.float32),
            pltpu.VMEM((CHUNK, 128), jnp.float32),
            pltpu.VMEM((CHUNK, 128), jnp.float32),
            pltpu.VMEM((L,), jnp.float32),
            pltpu.SemaphoreType.DMA,
        ],
    )


def _tc_body(yt_ref, p_ref, o_ref):
    b = pl.program_id(0)

    @pl.when(b == 0)
    def _():
        o_ref[0, 0] = jnp.sum(p_ref[...])

    yv = yt_ref[...]
    cols = b * CBLK + lax.broadcasted_iota(jnp.int32, (D, CBLK), 1)
    yv = jnp.where(cols < N_ROWS, yv, 0.0)
    o_ref[0, 0] += jnp.sum(yv * yv)


_tc_loss = pl.pallas_call(
    _tc_body,
    grid=(NBLK,),
    in_specs=[
        pl.BlockSpec((D, CBLK), lambda b: (0, b)),
        pl.BlockSpec((NW, L), lambda b: (0, 0)),
    ],
    out_specs=pl.BlockSpec(memory_space=pltpu.SMEM),
    out_shape=jax.ShapeDtypeStruct((1, 1), jnp.float32),
)


def kernel(Z, Y, interaction):
    u = interaction[:, 0].reshape(NW * NCH, CHUNK)
    i = interaction[:, 1].reshape(NW * NCH, CHUNK)
    r = interaction[:, 2].astype(jnp.float32)
    zp = Z.reshape(NP_ROWS, PACK * D)
    yp = Y.reshape(NP_ROWS, PACK * D)
    partials = _build_sc_partials()(zp, yp, u, i, r)
    loss = _tc_loss(Y.T, partials)
    return loss[0, 0]
